# Initial kernel scaffold; baseline (speedup 1.0000x reference)
import jax
import jax.numpy as jnp
from jax.experimental import pallas as pl


def kernel(pred_scores, pred_bboxes, anchor_points, gt_labels, gt_bboxes):
    raise NotImplementedError("write your pallas kernel here")


# speedup vs baseline: 16.9045x; 16.9045x over previous
import functools

import jax
import jax.numpy as jnp
from jax.experimental import pallas as pl
from jax.experimental.pallas import tpu as pltpu

_NUM_CLASSES = 80
_RADIUS = 2.5
_CAND_TOPK = 10
_IOU_W = 3.0
_CLS_W = 1.0
_BIG_I = 2**30


def _assign_kernel(ps_ref, pbt_ref, apt_ref, gl_ref, gb_ref,
                   lab_ref, bb_ref, sc_ref):
    N = ps_ref.shape[0]
    G = gl_ref.shape[1]
    C = _NUM_CLASSES

    s = ps_ref[...]
    p = jax.nn.sigmoid(s)
    logp = jnp.maximum(jnp.log(p), -100.0)
    log1mp = jnp.maximum(jnp.log(1.0 - p), -100.0)
    S = jnp.sum(log1mp, axis=1, keepdims=True)
    T = log1mp - logp

    labels_row = gl_ref[...]
    onehot = (labels_row.reshape(G, 1) ==
              jax.lax.broadcasted_iota(jnp.int32, (1, C), 1)).astype(jnp.float32)
    cls_sel = jax.lax.dot_general(
        onehot, T, (((1,), (1,)), ((), ())),
        precision=jax.lax.Precision.HIGHEST,
        preferred_element_type=jnp.float32)
    cls_cost = cls_sel - S.reshape(1, N)

    px1 = pbt_ref[0:1, :]
    py1 = pbt_ref[1:2, :]
    px2 = pbt_ref[2:3, :]
    py2 = pbt_ref[3:4, :]
    gx1 = gb_ref[:, 0:1]
    gy1 = gb_ref[:, 1:2]
    gx2 = gb_ref[:, 2:3]
    gy2 = gb_ref[:, 3:4]
    ltx = jnp.maximum(px1, gx1)
    lty = jnp.maximum(py1, gy1)
    rbx = jnp.minimum(px2, gx2)
    rby = jnp.minimum(py2, gy2)
    wx = jnp.maximum(rbx - ltx, 0.0)
    wy = jnp.maximum(rby - lty, 0.0)
    overlap = wx * wy
    area1 = (px2 - px1) * (py2 - py1)
    area2 = (gx2 - gx1) * (gy2 - gy1)
    union = area1 + area2 - overlap + 1e-6
    ious = overlap / union

    ax = apt_ref[0:1, :]
    ay = apt_ref[1:2, :]
    in_gt = (ax >= gx1) & (ax <= gx2) & (ay >= gy1) & (ay <= gy2)
    cx = (gx1 + gx2) / 2
    cy = (gy1 + gy2) / 2
    rx = _RADIUS * (gx2 - gx1)
    ry = _RADIUS * (gy2 - gy1)
    in_center = (ax >= cx - rx) & (ax <= cx + rx) & (ay >= cy - ry) & (ay <= cy + ry)
    inside = in_gt & in_center

    iou_cost = -jnp.log(ious)
    cost = _CLS_W * cls_cost + _IOU_W * iou_cost
    cost = cost + jnp.where(inside, 0.0, 1.0) * 1e10

    ious_in = ious * inside.astype(jnp.float32)
    num_cand = jnp.sum((ious_in > 0).astype(jnp.int32), axis=1, keepdims=True)
    ks = jnp.clip(num_cand, 1, _CAND_TOPK)

    iota_n = jax.lax.broadcasted_iota(jnp.int32, (G, N), 1)
    work = cost
    sel = jnp.zeros((G, N), dtype=jnp.bool_)
    for j in range(_CAND_TOPK):
        m = jnp.min(work, axis=1, keepdims=True)
        is_min = work == m
        idx = jnp.min(jnp.where(is_min, iota_n, _BIG_I),
                      axis=1, keepdims=True)
        pick = iota_n == idx
        sel = sel | (pick & (j < ks))
        work = jnp.where(pick, jnp.inf, work)

    iota_g = jax.lax.broadcasted_iota(jnp.int32, (G, N), 0)
    assigned = jnp.max(jnp.where(sel, iota_g, -1), axis=0, keepdims=True)
    pos = assigned >= 0

    mg = assigned == iota_g
    lab = jnp.sum(jnp.where(mg, labels_row.reshape(G, 1), 0), axis=0, keepdims=True)
    lab = jnp.where(pos, lab, C)
    iou_pa = jnp.sum(jnp.where(mg, ious, 0.0), axis=0, keepdims=True)
    val = jnp.where(pos, iou_pa, 0.0)

    assigned_t = assigned.reshape(N, 1)
    pos_t = assigned_t >= 0
    lab_t = lab.reshape(N, 1)
    val_t = val.reshape(N, 1)

    lab_ref[...] = lab_t

    mg_t = assigned_t == jax.lax.broadcasted_iota(jnp.int32, (1, G), 1)
    mg_tf = mg_t.astype(jnp.float32)
    bb = jnp.concatenate(
        [jnp.sum(jnp.where(mg_t, gb_ref[:, i:i + 1].reshape(1, G), 0.0),
                 axis=1, keepdims=True) for i in range(4)], axis=1)
    bb_ref[...] = jnp.where(pos_t, bb, 0.0)
    del mg_tf

    iota_c = jax.lax.broadcasted_iota(jnp.int32, (1, C + 1), 1)
    sc_ref[...] = jnp.where(lab_t == iota_c, val_t, 0.0)


@jax.jit
def kernel(pred_scores, pred_bboxes, anchor_points, gt_labels, gt_bboxes):
    N, C = pred_scores.shape
    G = gt_labels.shape[0]
    out = pl.pallas_call(
        _assign_kernel,
        out_shape=(
            jax.ShapeDtypeStruct((N, 1), jnp.int32),
            jax.ShapeDtypeStruct((N, 4), jnp.float32),
            jax.ShapeDtypeStruct((N, C + 1), jnp.float32),
        ),
        compiler_params=pltpu.CompilerParams(
            vmem_limit_bytes=100 * 1024 * 1024,
        ),
    )(pred_scores, pred_bboxes.T, anchor_points.T, gt_labels.reshape(1, G),
      gt_bboxes)
    lab, bb, sc = out
    return lab.reshape(N), bb, sc
